# Initial kernel scaffold; baseline (speedup 1.0000x reference)
import jax
import jax.numpy as jnp
from jax.experimental import pallas as pl


def kernel(X0, X1, X2, L0_idx, L0_val, L1_idx, L1_val, L1u_idx, L1d_idx, L2_idx, L2_val, batch0, batch1, batch2, params):
    raise NotImplementedError("write your pallas kernel here")


# speedup vs baseline: 6.7662x; 6.7662x over previous
import functools

import jax
import jax.numpy as jnp
from jax import lax
from jax.experimental import pallas as pl
from jax.experimental.pallas import tpu as pltpu
from jax.experimental.pallas import tpu_sc as plsc

_NB = 32

_NW = 32
_CH = 128
_GRAN = _NW * _CH


@functools.lru_cache(maxsize=None)
def _sc_gather_kernel(V, D, EPAD):
    rows_w = EPAD // _NW
    nch = rows_w // _CH
    mesh = plsc.VectorSubcoreMesh(core_axis_name="c", subcore_axis_name="s")

    @functools.partial(
        pl.kernel,
        mesh=mesh,
        compiler_params=pltpu.CompilerParams(use_tc_tiling_on_sc=False),
        out_type=jax.ShapeDtypeStruct((EPAD, D), jnp.float32),
        scratch_types=[
            pltpu.VMEM((rows_w,), jnp.int32),
            pltpu.VMEM((_CH, D), jnp.float32),
            pltpu.SemaphoreType.DMA,
        ],
    )
    def k(table_hbm, idx_hbm, out_hbm, idx_v, rows_v, sem):
        wid = lax.axis_index("s") * 2 + lax.axis_index("c")
        base = wid * rows_w
        pltpu.sync_copy(idx_hbm.at[pl.ds(base, rows_w)], idx_v)

        def body(c, carry):
            pltpu.async_copy(
                table_hbm.at[idx_v.at[pl.ds(c * _CH, _CH)]], rows_v, sem
            ).wait()
            pltpu.sync_copy(rows_v, out_hbm.at[pl.ds(base + c * _CH, _CH)])
            return carry

        lax.fori_loop(0, nch, body, 0)

    return k


def _pad_idx(idx):
    e = idx.shape[0]
    epad = -(-e // _GRAN) * _GRAN
    return jnp.pad(idx, (0, epad - e)), e


def _sc_gather(table, idx_pad, e, d_used):
    V, D = table.shape
    out = _sc_gather_kernel(V, D, idx_pad.shape[0])(table, idx_pad)
    return out[:e, :d_used]


def _mm_body(x_ref, w_ref, o_ref):
    o_ref[...] = jnp.dot(x_ref[...], w_ref[...],
                         preferred_element_type=jnp.float32)


def _mm(x, w):
    n, kdim = x.shape
    m = w.shape[1]
    blk = 512
    npad = -(-n // blk) * blk
    xp = jnp.pad(x, ((0, npad - n), (0, 128 - kdim)))
    wp = jnp.pad(w, ((0, 128 - kdim), (0, 128 - m)))
    out = pl.pallas_call(
        _mm_body,
        grid=(npad // blk,),
        in_specs=[
            pl.BlockSpec((blk, 128), lambda i: (i, 0)),
            pl.BlockSpec((128, 128), lambda i: (0, 0)),
        ],
        out_specs=pl.BlockSpec((blk, 128), lambda i: (i, 0)),
        out_shape=jax.ShapeDtypeStruct((npad, 128), jnp.float32),
    )(xp, wp)
    return out[:n, :m]


def _pack32(h_aug):
    n, c = h_aug.shape
    return jnp.pad(h_aug, ((0, 0), (0, 32 - c)))


def _gat(table, h, srcp, dstp, src_raw, dst_raw, e_cnt, p):
    n = h.shape[0]
    fs = h.shape[1]
    g_src = _sc_gather(table, srcp, e_cnt, fs + 1)
    g_dst = _sc_gather(table, dstp, e_cnt, fs + 2)
    h_src = g_src[:, :fs]
    logit = jax.nn.leaky_relu(g_src[:, fs] + g_dst[:, fs + 1], 0.2)
    mseg = jax.ops.segment_max(logit, dst_raw, num_segments=n)
    m_tab = jnp.pad(mseg[:, None], ((0, 0), (0, 15)))
    m_dst = _sc_gather(m_tab, dstp, e_cnt, 1)[:, 0]
    ex = jnp.exp(logit - m_dst)
    den = jax.ops.segment_sum(ex, dst_raw, num_segments=n)
    d_tab = jnp.pad(den[:, None], ((0, 0), (0, 15)))
    den_dst = _sc_gather(d_tab, dstp, e_cnt, 1)[:, 0]
    coef = ex / (den_dst + 1e-16)
    return jax.ops.segment_sum(coef[:, None] * h_src, dst_raw,
                               num_segments=n) + p['b']


def _san(x, lu, ld, pp, p_val, prm):
    n = x.shape[0]
    xp = _mm(x, prm['p_W']) + prm['p_b']
    fs = xp.shape[1]
    srcp, dstp, src_raw, dst_raw, e_cnt = pp
    xp_tab = _pack32(xp)
    g = _sc_gather(xp_tab, dstp, e_cnt, fs)
    out = jax.ops.segment_sum(p_val[:, None] * g, src_raw, num_segments=n)

    if lu is not None or ld is not None:
        gp = prm['gat']
        w_aug = jnp.concatenate(
            [gp['W'], (gp['W'] @ gp['a_src'])[:, None],
             (gp['W'] @ gp['a_dst'])[:, None]], axis=1)
        h_aug = _mm(x, w_aug)
        h = h_aug[:, :fs]
        table = _pack32(h_aug)
        if lu is not None:
            out = out + _gat(table, h, *lu, gp)
        if ld is not None:
            out = out + _gat(table, h, *ld, gp)
    return out


def _gmp(x, batch, nb):
    s = jax.ops.segment_sum(x, batch, num_segments=nb)
    c = jax.ops.segment_sum(jnp.ones((x.shape[0],), x.dtype), batch,
                            num_segments=nb)
    return s / jnp.maximum(c, 1.0)[:, None]


def _edges(idx):
    srcp, e = _pad_idx(idx[0])
    dstp, _ = _pad_idx(idx[1])
    return (srcp, dstp, idx[0], idx[1], e)


def kernel(X0, X1, X2, L0_idx, L0_val, L1_idx, L1_val, L1u_idx, L1d_idx,
           L2_idx, L2_val, batch0, batch1, batch2, params):
    r = jax.nn.relu
    e0 = _edges(L0_idx)
    e1 = _edges(L1_idx)
    e1u = _edges(L1u_idx)
    e1d = _edges(L1d_idx)
    e2 = _edges(L2_idx)

    x0_1 = r(_san(X0, None, e0, e0, L0_val, params['l0_1']))
    x0_2 = r(_san(x0_1, None, e0, e0, L0_val, params['l0_2']))
    x0_3 = r(_san(x0_2, None, e0, e0, L0_val, params['l0_3']))
    x0_4 = _mm(jnp.concatenate([x0_1, x0_2, x0_3], axis=1),
               params['l0_4_W']) + params['l0_4_b']
    x0 = _gmp(x0_4, batch0, _NB)

    x1_1 = r(_san(X1, e1u, e1d, e1, L1_val, params['l1_1']))
    x1_2 = r(_san(x1_1, e1u, e1d, e1, L1_val, params['l1_2']))
    x1_3 = r(_san(x1_2, e1u, e1d, e1, L1_val, params['l1_3']))
    x1_4 = _mm(jnp.concatenate([x1_1, x1_2, x1_3], axis=1),
               params['l1_4_W']) + params['l1_4_b']
    x1 = _gmp(x1_4, batch1, _NB)

    x2_1 = r(_san(X2, e2, None, e2, L2_val, params['l2_1']))
    x2_2 = r(_san(x2_1, e2, None, e2, L2_val, params['l2_2']))
    x2_3 = r(_san(x2_2, e2, None, e2, L2_val, params['l2_3']))
    x2_4 = _mm(jnp.concatenate([x2_1, x2_2, x2_3], axis=1),
               params['l2_4_W']) + params['l2_4_b']
    x2 = _gmp(x2_4, batch2, _NB)

    x = jnp.concatenate([x0, x1, x2], axis=1)
    logits = _mm(x, params['comb_W']) + params['comb_b']
    return jax.nn.softmax(logits, axis=1)


# speedup vs baseline: 6.8329x; 1.0099x over previous
import functools

import jax
import jax.numpy as jnp
from jax import lax
from jax.experimental import pallas as pl
from jax.experimental.pallas import tpu as pltpu
from jax.experimental.pallas import tpu_sc as plsc

_NB = 32

_NW = 32
_CH = 128
_GRAN = _NW * _CH


@functools.lru_cache(maxsize=None)
def _sc_gather_kernel(V, D, EPAD):
    rows_w = EPAD // _NW
    nch = rows_w // _CH
    mesh = plsc.VectorSubcoreMesh(core_axis_name="c", subcore_axis_name="s")

    @functools.partial(
        pl.kernel,
        mesh=mesh,
        compiler_params=pltpu.CompilerParams(use_tc_tiling_on_sc=False),
        out_type=jax.ShapeDtypeStruct((EPAD, D), jnp.float32),
        scratch_types=[
            pltpu.VMEM((rows_w,), jnp.int32),
            pltpu.VMEM((_CH, D), jnp.float32),
            pltpu.SemaphoreType.DMA,
        ],
    )
    def k(table_hbm, idx_hbm, out_hbm, idx_v, rows_v, sem):
        wid = lax.axis_index("s") * 2 + lax.axis_index("c")
        base = wid * rows_w
        pltpu.sync_copy(idx_hbm.at[pl.ds(base, rows_w)], idx_v)

        def body(c, carry):
            pltpu.async_copy(
                table_hbm.at[idx_v.at[pl.ds(c * _CH, _CH)]], rows_v, sem
            ).wait()
            pltpu.sync_copy(rows_v, out_hbm.at[pl.ds(base + c * _CH, _CH)])
            return carry

        lax.fori_loop(0, nch, body, 0)

    return k


def _pad_idx(idx):
    e = idx.shape[0]
    epad = -(-e // _GRAN) * _GRAN
    return jnp.pad(idx, (0, epad - e)), e


def _sc_gather(table, idx_pad, e, d_used):
    V, D = table.shape
    out = _sc_gather_kernel(V, D, idx_pad.shape[0])(table, idx_pad)
    return out[:e, :d_used]


def _mm_body(x_ref, w_ref, o_ref):
    o_ref[...] = jnp.dot(x_ref[...], w_ref[...],
                         preferred_element_type=jnp.float32)


def _mm(x, w):
    n, kdim = x.shape
    m = w.shape[1]
    blk = 512
    npad = -(-n // blk) * blk
    xp = jnp.pad(x, ((0, npad - n), (0, 128 - kdim)))
    wp = jnp.pad(w, ((0, 128 - kdim), (0, 128 - m)))
    out = pl.pallas_call(
        _mm_body,
        grid=(npad // blk,),
        in_specs=[
            pl.BlockSpec((blk, 128), lambda i: (i, 0)),
            pl.BlockSpec((128, 128), lambda i: (0, 0)),
        ],
        out_specs=pl.BlockSpec((blk, 128), lambda i: (i, 0)),
        out_shape=jax.ShapeDtypeStruct((npad, 128), jnp.float32),
    )(xp, wp)
    return out[:n, :m]


def _pack32(h_aug):
    n, c = h_aug.shape
    return jnp.pad(h_aug, ((0, 0), (0, 32 - c)))


def _gat(table, h, srcp, dstp, src_raw, dst_raw, e_cnt, p):
    n = h.shape[0]
    fs = h.shape[1]
    g_src = _sc_gather(table, srcp, e_cnt, fs + 1)
    d_tab16 = jnp.pad(table[:, fs + 1:fs + 2], ((0, 0), (0, 15)))
    d_dst = _sc_gather(d_tab16, dstp, e_cnt, 1)[:, 0]
    h_src = g_src[:, :fs]
    logit = jax.nn.leaky_relu(g_src[:, fs] + d_dst, 0.2)
    mseg = jax.ops.segment_max(logit, dst_raw, num_segments=n)
    m_tab = jnp.pad(mseg[:, None], ((0, 0), (0, 15)))
    m_dst = _sc_gather(m_tab, dstp, e_cnt, 1)[:, 0]
    ex = jnp.exp(logit - m_dst)
    den = jax.ops.segment_sum(ex, dst_raw, num_segments=n)
    d_tab = jnp.pad(den[:, None], ((0, 0), (0, 15)))
    den_dst = _sc_gather(d_tab, dstp, e_cnt, 1)[:, 0]
    coef = ex / (den_dst + 1e-16)
    return jax.ops.segment_sum(coef[:, None] * h_src, dst_raw,
                               num_segments=n) + p['b']


def _san(x, lu, ld, pp, p_val, prm):
    n = x.shape[0]
    xp = _mm(x, prm['p_W']) + prm['p_b']
    fs = xp.shape[1]
    srcp, dstp, src_raw, dst_raw, e_cnt = pp
    xp_tab = _pack32(xp)
    g = _sc_gather(xp_tab, dstp, e_cnt, fs)
    out = jax.ops.segment_sum(p_val[:, None] * g, src_raw, num_segments=n)

    if lu is not None or ld is not None:
        gp = prm['gat']
        w_aug = jnp.concatenate(
            [gp['W'], (gp['W'] @ gp['a_src'])[:, None],
             (gp['W'] @ gp['a_dst'])[:, None]], axis=1)
        h_aug = _mm(x, w_aug)
        h = h_aug[:, :fs]
        table = _pack32(h_aug)
        if lu is not None:
            out = out + _gat(table, h, *lu, gp)
        if ld is not None:
            out = out + _gat(table, h, *ld, gp)
    return out


def _gmp(x, batch, nb):
    s = jax.ops.segment_sum(x, batch, num_segments=nb)
    c = jax.ops.segment_sum(jnp.ones((x.shape[0],), x.dtype), batch,
                            num_segments=nb)
    return s / jnp.maximum(c, 1.0)[:, None]


def _edges(idx):
    srcp, e = _pad_idx(idx[0])
    dstp, _ = _pad_idx(idx[1])
    return (srcp, dstp, idx[0], idx[1], e)


def kernel(X0, X1, X2, L0_idx, L0_val, L1_idx, L1_val, L1u_idx, L1d_idx,
           L2_idx, L2_val, batch0, batch1, batch2, params):
    r = jax.nn.relu
    e0 = _edges(L0_idx)
    e1 = _edges(L1_idx)
    e1u = _edges(L1u_idx)
    e1d = _edges(L1d_idx)
    e2 = _edges(L2_idx)

    x0_1 = r(_san(X0, None, e0, e0, L0_val, params['l0_1']))
    x0_2 = r(_san(x0_1, None, e0, e0, L0_val, params['l0_2']))
    x0_3 = r(_san(x0_2, None, e0, e0, L0_val, params['l0_3']))
    x0_4 = _mm(jnp.concatenate([x0_1, x0_2, x0_3], axis=1),
               params['l0_4_W']) + params['l0_4_b']
    x0 = _gmp(x0_4, batch0, _NB)

    x1_1 = r(_san(X1, e1u, e1d, e1, L1_val, params['l1_1']))
    x1_2 = r(_san(x1_1, e1u, e1d, e1, L1_val, params['l1_2']))
    x1_3 = r(_san(x1_2, e1u, e1d, e1, L1_val, params['l1_3']))
    x1_4 = _mm(jnp.concatenate([x1_1, x1_2, x1_3], axis=1),
               params['l1_4_W']) + params['l1_4_b']
    x1 = _gmp(x1_4, batch1, _NB)

    x2_1 = r(_san(X2, e2, None, e2, L2_val, params['l2_1']))
    x2_2 = r(_san(x2_1, e2, None, e2, L2_val, params['l2_2']))
    x2_3 = r(_san(x2_2, e2, None, e2, L2_val, params['l2_3']))
    x2_4 = _mm(jnp.concatenate([x2_1, x2_2, x2_3], axis=1),
               params['l2_4_W']) + params['l2_4_b']
    x2 = _gmp(x2_4, batch2, _NB)

    x = jnp.concatenate([x0, x1, x2], axis=1)
    logits = _mm(x, params['comb_W']) + params['comb_b']
    return jax.nn.softmax(logits, axis=1)
